# baseline (speedup 1.0000x reference)
import functools

import jax
import jax.numpy as jnp
from jax.experimental import pallas as pl

E = 16
TOPK = 2
BLK = 1024


def _router_block(hs_ref, w_ref, weights_ref, topw_ref, tope_ref):
    hs = hs_ref[...]
    w = w_ref[...]
    logits = jax.lax.dot_general(
        hs, w, (((1,), (1,)), ((), ())), preferred_element_type=jnp.float32
    )
    m = jnp.max(logits, axis=1, keepdims=True)
    ex = jnp.exp(logits - m)
    probs = ex / jnp.sum(ex, axis=1, keepdims=True)

    lanes = jax.lax.broadcasted_iota(jnp.int32, probs.shape, 1)
    m1 = jnp.max(probs, axis=1, keepdims=True)
    i1 = jnp.min(jnp.where(probs == m1, lanes, E), axis=1, keepdims=True)
    masked = jnp.where(lanes == i1, -1.0, probs)
    m2 = jnp.max(masked, axis=1, keepdims=True)
    i2 = jnp.min(jnp.where(masked == m2, lanes, E), axis=1, keepdims=True)

    denom = m1 + m2
    weights_ref[...] = probs
    topw_ref[...] = jnp.concatenate([m1 / denom, m2 / denom], axis=1)
    tope_ref[...] = jnp.concatenate([i1, i2], axis=1)


@functools.partial(jax.jit, static_argnames=("interpret",))
def _router(hs2d, W, interpret=False):
    n = hs2d.shape[0]
    h = hs2d.shape[1]
    grid = (n // BLK,)
    return pl.pallas_call(
        _router_block,
        grid=grid,
        in_specs=[
            pl.BlockSpec((BLK, h), lambda i: (i, 0)),
            pl.BlockSpec((E, h), lambda i: (0, 0)),
        ],
        out_specs=[
            pl.BlockSpec((BLK, E), lambda i: (i, 0)),
            pl.BlockSpec((BLK, TOPK), lambda i: (i, 0)),
            pl.BlockSpec((BLK, TOPK), lambda i: (i, 0)),
        ],
        out_shape=[
            jax.ShapeDtypeStruct((n, E), jnp.float32),
            jax.ShapeDtypeStruct((n, TOPK), jnp.float32),
            jax.ShapeDtypeStruct((n, TOPK), jnp.int32),
        ],
        interpret=interpret,
    )(hs2d, W)


def kernel(hidden_states, W):
    hs2d = hidden_states.reshape(-1, hidden_states.shape[-1])
    weights, top_weights, top_experts = _router(hs2d, W)
    weights = weights.astype(hidden_states.dtype)
    top_weights = top_weights.astype(hidden_states.dtype)
    return (weights, top_weights, top_experts)


# speedup vs baseline: 1.6690x; 1.6690x over previous
import functools

import jax
import jax.numpy as jnp
from jax.experimental import pallas as pl

E = 16
TOPK = 2
BLK = 1024


def _router_block(hs_ref, w_ref, weights_ref, topw_ref, tope_ref):
    hs = hs_ref[...]
    w = w_ref[...]
    lt = jax.lax.dot_general(
        w, hs, (((1,), (1,)), ((), ())), preferred_element_type=jnp.float32
    )
    m1 = jnp.max(lt, axis=0, keepdims=True)
    ex = jnp.exp(lt - m1)
    s = jnp.sum(ex, axis=0, keepdims=True)
    weights_ref[...] = ex * (1.0 / s)

    rows = jax.lax.broadcasted_iota(jnp.int32, lt.shape, 0)
    i1 = jnp.min(jnp.where(lt == m1, rows, E), axis=0, keepdims=True)
    masked = jnp.where(rows == i1, -jnp.inf, lt)
    m2 = jnp.max(masked, axis=0, keepdims=True)
    i2 = jnp.min(jnp.where(masked == m2, rows, E), axis=0, keepdims=True)

    e2 = jnp.exp(m2 - m1)
    tw1 = 1.0 / (1.0 + e2)
    topw_ref[...] = jnp.concatenate([tw1, e2 * tw1], axis=0)
    tope_ref[...] = jnp.concatenate([i1, i2], axis=0)


@functools.partial(jax.jit, static_argnames=("interpret",))
def _router(hs2d, W, interpret=False):
    n = hs2d.shape[0]
    h = hs2d.shape[1]
    grid = (n // BLK,)
    return pl.pallas_call(
        _router_block,
        grid=grid,
        in_specs=[
            pl.BlockSpec((BLK, h), lambda i: (i, 0)),
            pl.BlockSpec((E, h), lambda i: (0, 0)),
        ],
        out_specs=[
            pl.BlockSpec((E, BLK), lambda i: (0, i)),
            pl.BlockSpec((TOPK, BLK), lambda i: (0, i)),
            pl.BlockSpec((TOPK, BLK), lambda i: (0, i)),
        ],
        out_shape=[
            jax.ShapeDtypeStruct((E, n), jnp.float32),
            jax.ShapeDtypeStruct((TOPK, n), jnp.float32),
            jax.ShapeDtypeStruct((TOPK, n), jnp.int32),
        ],
        interpret=interpret,
    )(hs2d, W)


def kernel(hidden_states, W):
    hs2d = hidden_states.reshape(-1, hidden_states.shape[-1])
    weights_t, top_weights_t, top_experts_t = _router(hs2d, W)
    weights = weights_t.T.astype(hidden_states.dtype)
    top_weights = top_weights_t.T.astype(hidden_states.dtype)
    top_experts = top_experts_t.T
    return (weights, top_weights, top_experts)
